# baseline (device time: 23430 ns/iter reference)
import jax
import jax.numpy as jnp
from jax import lax
from jax.experimental import pallas as pl
from jax.experimental.pallas import tpu as pltpu

N_DEV = 4
H_GLOBAL = 512
W = 128
N_NORM = H_GLOBAL * W
EPS = 1e-5


def kernel(x, Wp):
    b, h_per, w, c = x.shape
    c_out = Wp.shape[1]

    xt = pltpu.with_memory_space_constraint(
        jnp.transpose(x, (0, 1, 3, 2)), pltpu.HBM
    )
    Wp = pltpu.with_memory_space_constraint(Wp, pltpu.HBM)

    def body(xt_ref, wp_ref, out_ref, stats_ref, xbuf_ref, wp_buf_ref,
             send_sems, recv_sems, in_sems, wp_sem):
        my = lax.axis_index("i")

        barrier_sem = pltpu.get_barrier_semaphore()
        for d in range(1, N_DEV):
            pl.semaphore_signal(
                barrier_sem, inc=1,
                device_id=((my + d) % N_DEV,),
                device_id_type=pl.DeviceIdType.MESH,
            )

        wp_dma = pltpu.make_async_copy(wp_ref, wp_buf_ref, wp_sem)
        wp_dma.start()

        hh2 = h_per // 2
        dmas = []
        for k in range(4):
            bb, hk = divmod(k, 2)
            dma = pltpu.make_async_copy(
                xt_ref.at[bb, pl.ds(hk * hh2, hh2)],
                xbuf_ref.at[bb, pl.ds(hk * hh2, hh2)],
                in_sems.at[k],
            )
            dma.start()
            dmas.append(dma)

        s_parts = []
        sq_parts = []
        for k in range(4):
            bb, hk = divmod(k, 2)
            dmas[k].wait()
            ck = xbuf_ref[bb, hk * hh2:(hk + 1) * hh2]
            s_parts.append(jnp.sum(jnp.sum(ck, axis=2), axis=0))
            sq_parts.append(jnp.sum(jnp.sum(ck * ck, axis=2), axis=0))
        s0 = (s_parts[0] + s_parts[1])[None, :]
        s1 = (s_parts[2] + s_parts[3])[None, :]
        q0 = (sq_parts[0] + sq_parts[1])[None, :]
        q1 = (sq_parts[2] + sq_parts[3])[None, :]
        stats_ref[0, :, :] = jnp.concatenate(
            [s0, s1, q0, q1, jnp.zeros((8 - 2 * b, c), jnp.float32)], axis=0
        )

        pl.semaphore_wait(barrier_sem, N_DEV - 1)

        rdmas = []
        for d in range(1, N_DEV):
            rdma = pltpu.make_async_remote_copy(
                src_ref=stats_ref.at[0],
                dst_ref=stats_ref.at[N_DEV - d],
                send_sem=send_sems.at[d - 1],
                recv_sem=recv_sems.at[N_DEV - d],
                device_id=((my + d) % N_DEV,),
                device_id_type=pl.DeviceIdType.MESH,
            )
            rdma.start()
            rdmas.append(rdma)

        for rdma in rdmas:
            rdma.wait()

        total = jnp.sum(stats_ref[...], axis=0)

        inv_n = jnp.float32(1.0 / N_NORM)
        mean = total[0:b, :] * inv_n
        ex2 = total[b:2 * b, :] * inv_n
        rstd = lax.rsqrt(ex2 - mean * mean + EPS)

        xtv = xbuf_ref[...]
        hh = (xtv - mean[:, None, :, None]) * rstd[:, None, :, None]
        a4 = hh * (1.0 / (1.0 + jnp.exp(-hh)))

        wp_dma.wait()
        wpv = wp_buf_ref[...]
        ob = lax.dot_general(
            a4.astype(jnp.bfloat16), wpv.astype(jnp.bfloat16),
            (((2,), (0,)), ((), ())),
            preferred_element_type=jnp.float32,
        )
        out_ref[...] = ob

    return pl.pallas_call(
        body,
        out_shape=jax.ShapeDtypeStruct((b, h_per, w, c_out), jnp.float32),
        in_specs=[
            pl.BlockSpec(memory_space=pl.ANY),
            pl.BlockSpec(memory_space=pl.ANY),
        ],
        out_specs=pl.BlockSpec(memory_space=pltpu.VMEM),
        scratch_shapes=[
            pltpu.VMEM((N_DEV, 8, 64), jnp.float32),
            pltpu.VMEM((b, h_per, c, w), jnp.float32),
            pltpu.VMEM((c, c_out), jnp.float32),
            pltpu.SemaphoreType.DMA((N_DEV - 1,)),
            pltpu.SemaphoreType.DMA((N_DEV,)),
            pltpu.SemaphoreType.DMA((4,)),
            pltpu.SemaphoreType.DMA(()),
        ],
        compiler_params=pltpu.CompilerParams(collective_id=0),
    )(xt, Wp)


# device time: 20750 ns/iter; 1.1292x vs baseline; 1.1292x over previous
import jax
import jax.numpy as jnp
from jax import lax
from jax.experimental import pallas as pl
from jax.experimental.pallas import tpu as pltpu

N_DEV = 4
H_GLOBAL = 512
W = 128
N_NORM = H_GLOBAL * W
EPS = 1e-5


def kernel(x, Wp):
    b, h_per, w, c = x.shape
    c_out = Wp.shape[1]

    xt = pltpu.with_memory_space_constraint(
        jnp.transpose(x, (0, 1, 3, 2)), pltpu.HBM
    )

    def body(xt_ref, wp_ref, out_ref, stats_ref, xbuf_ref, send_sems,
             recv_sems, in_sems):
        my = lax.axis_index("i")

        barrier_sem = pltpu.get_barrier_semaphore()
        for d in range(1, N_DEV):
            pl.semaphore_signal(
                barrier_sem, inc=1,
                device_id=((my + d) % N_DEV,),
                device_id_type=pl.DeviceIdType.MESH,
            )

        hh2 = h_per // 2
        dmas = []
        for k in range(4):
            bb, hk = divmod(k, 2)
            dma = pltpu.make_async_copy(
                xt_ref.at[bb, pl.ds(hk * hh2, hh2)],
                xbuf_ref.at[bb, pl.ds(hk * hh2, hh2)],
                in_sems.at[k],
            )
            dma.start()
            dmas.append(dma)

        s_parts = []
        sq_parts = []
        for k in range(4):
            bb, hk = divmod(k, 2)
            dmas[k].wait()
            ck = xbuf_ref[bb, hk * hh2:(hk + 1) * hh2]
            s_parts.append(jnp.sum(jnp.sum(ck, axis=2), axis=0))
            sq_parts.append(jnp.sum(jnp.sum(ck * ck, axis=2), axis=0))
        s0 = (s_parts[0] + s_parts[1])[None, :]
        s1 = (s_parts[2] + s_parts[3])[None, :]
        q0 = (sq_parts[0] + sq_parts[1])[None, :]
        q1 = (sq_parts[2] + sq_parts[3])[None, :]
        stats_ref[0, :, :] = jnp.concatenate(
            [s0, s1, q0, q1, jnp.zeros((8 - 2 * b, c), jnp.float32)], axis=0
        )

        pl.semaphore_wait(barrier_sem, N_DEV - 1)

        rdmas = []
        for d in range(1, N_DEV):
            rdma = pltpu.make_async_remote_copy(
                src_ref=stats_ref.at[0],
                dst_ref=stats_ref.at[N_DEV - d],
                send_sem=send_sems.at[d - 1],
                recv_sem=recv_sems.at[N_DEV - d],
                device_id=((my + d) % N_DEV,),
                device_id_type=pl.DeviceIdType.MESH,
            )
            rdma.start()
            rdmas.append(rdma)

        for rdma in rdmas:
            rdma.wait()

        total = jnp.sum(stats_ref[...], axis=0)

        inv_n = jnp.float32(1.0 / N_NORM)
        mean = total[0:b, :] * inv_n
        ex2 = total[b:2 * b, :] * inv_n
        rstd = lax.rsqrt(ex2 - mean * mean + EPS)

        xtv = xbuf_ref[...]
        hh = (xtv - mean[:, None, :, None]) * rstd[:, None, :, None]
        a4 = hh * (1.0 / (1.0 + jnp.exp(-hh)))

        wpv = wp_ref[...]
        ob = lax.dot_general(
            a4.astype(jnp.bfloat16), wpv.astype(jnp.bfloat16),
            (((2,), (0,)), ((), ())),
            preferred_element_type=jnp.float32,
        )
        out_ref[...] = ob

    return pl.pallas_call(
        body,
        out_shape=jax.ShapeDtypeStruct((b, h_per, w, c_out), jnp.float32),
        in_specs=[
            pl.BlockSpec(memory_space=pl.ANY),
            pl.BlockSpec(memory_space=pltpu.VMEM),
        ],
        out_specs=pl.BlockSpec(memory_space=pltpu.VMEM),
        scratch_shapes=[
            pltpu.VMEM((N_DEV, 8, 64), jnp.float32),
            pltpu.VMEM((b, h_per, c, w), jnp.float32),
            pltpu.SemaphoreType.DMA((N_DEV - 1,)),
            pltpu.SemaphoreType.DMA((N_DEV,)),
            pltpu.SemaphoreType.DMA((4,)),
        ],
        compiler_params=pltpu.CompilerParams(collective_id=0),
    )(xt, Wp)
